# Initial kernel scaffold; baseline (speedup 1.0000x reference)
import jax
import jax.numpy as jnp
from jax.experimental import pallas as pl


def kernel(hist_ids, wish_ids, bid, auth, lang, tags, dense, book_emb, auth_emb, lang_emb, tag_emb, W1, b1, W2, b2, Wu1, bu1, Wu2, bu2, Wu3, bu3):
    raise NotImplementedError("write your pallas kernel here")


# speedup vs baseline: 1.1042x; 1.1042x over previous
import functools

import jax
import jax.numpy as jnp
from jax import lax
from jax.experimental import pallas as pl
from jax.experimental.pallas import tpu as pltpu
from jax.experimental.pallas import tpu_sc as plsc

B = 4096
ED = 64
NHIST = 50
NWISH = 20
NTAGS = 10
NCORES = 2
NSUB = 16
NW = NCORES * NSUB
ROWS_W = B // NW
HC = 8
NCH = ROWS_W // HC
GSUB = 80

_f32 = jnp.float32


def _sc_body(hist_hbm, wish_hbm, tags_hbm, bid_hbm, auth_hbm, lang_hbm,
             book_hbm, aemb_hbm, lemb_hbm, temb_hbm,
             u0_hbm, ip_hbm,
             idx_h, idx_w, idx_t, idx_b, idx_a, idx_l,
             rows_h, rows_w, rows_t, rows_b, rows_a, rows_l,
             out_u, out_i, sem):
    wid = lax.axis_index("s") * NCORES + lax.axis_index("c")
    rbase = wid * ROWS_W

    pltpu.sync_copy(hist_hbm.at[pl.ds(rbase * NHIST, ROWS_W * NHIST)], idx_h)
    pltpu.sync_copy(wish_hbm.at[pl.ds(rbase * NWISH, ROWS_W * NWISH)], idx_w)
    pltpu.sync_copy(tags_hbm.at[pl.ds(rbase * NTAGS, ROWS_W * NTAGS)], idx_t)
    pltpu.sync_copy(bid_hbm.at[pl.ds(rbase, ROWS_W)], idx_b)
    pltpu.sync_copy(auth_hbm.at[pl.ds(rbase, ROWS_W)], idx_a)
    pltpu.sync_copy(lang_hbm.at[pl.ds(rbase, ROWS_W)], idx_l)

    def chunk(c, carry):
        hoff = c * (HC * NHIST)
        woff = c * (HC * NWISH)
        toff = c * (HC * NTAGS)
        boff = c * HC
        cps = []
        for k in range(HC * NHIST // GSUB):
            cps.append(pltpu.async_copy(
                book_hbm.at[idx_h.at[pl.ds(hoff + k * GSUB, GSUB)]],
                rows_h.at[pl.ds(k * GSUB, GSUB)], sem))
        for k in range(HC * NWISH // GSUB):
            cps.append(pltpu.async_copy(
                book_hbm.at[idx_w.at[pl.ds(woff + k * GSUB, GSUB)]],
                rows_w.at[pl.ds(k * GSUB, GSUB)], sem))
        cps.append(pltpu.async_copy(
            temb_hbm.at[idx_t.at[pl.ds(toff, HC * NTAGS)]], rows_t, sem))
        cps.append(pltpu.async_copy(
            book_hbm.at[idx_b.at[pl.ds(boff, HC)]], rows_b, sem))
        cps.append(pltpu.async_copy(
            aemb_hbm.at[idx_a.at[pl.ds(boff, HC)]], rows_a, sem))
        cps.append(pltpu.async_copy(
            lemb_hbm.at[idx_l.at[pl.ds(boff, HC)]], rows_l, sem))
        for cp in cps:
            cp.wait()

        for r in range(HC):
            for g in range(ED // 16):
                s = pl.ds(g * 16, 16)
                acch = rows_h[r * NHIST, s]
                for j in range(1, NHIST):
                    acch = acch + rows_h[r * NHIST + j, s]
                accw = rows_w[r * NWISH, s]
                for j in range(1, NWISH):
                    accw = accw + rows_w[r * NWISH + j, s]
                out_u[r, s] = acch * (1.0 / NHIST) + accw * (1.0 / NWISH)
                acct = rows_t[r * NTAGS, s]
                for j in range(1, NTAGS):
                    acct = acct + rows_t[r * NTAGS + j, s]
                out_i[r, s] = (rows_b[r, s] + rows_a[r, s] + rows_l[r, s]
                               + acct * (1.0 / NTAGS))

        pltpu.sync_copy(out_u, u0_hbm.at[pl.ds(rbase + c * HC, HC)])
        pltpu.sync_copy(out_i, ip_hbm.at[pl.ds(rbase + c * HC, HC)])
        return carry

    lax.fori_loop(0, NCH, chunk, 0)


_sc_gather_pool = functools.partial(
    pl.kernel,
    out_type=(jax.ShapeDtypeStruct((B, ED), _f32),
              jax.ShapeDtypeStruct((B, ED), _f32)),
    mesh=plsc.VectorSubcoreMesh(core_axis_name="c", subcore_axis_name="s"),
    scratch_types=[
        pltpu.VMEM((ROWS_W * NHIST,), jnp.int32),
        pltpu.VMEM((ROWS_W * NWISH,), jnp.int32),
        pltpu.VMEM((ROWS_W * NTAGS,), jnp.int32),
        pltpu.VMEM((ROWS_W,), jnp.int32),
        pltpu.VMEM((ROWS_W,), jnp.int32),
        pltpu.VMEM((ROWS_W,), jnp.int32),
        pltpu.VMEM((HC * NHIST, ED), _f32),
        pltpu.VMEM((HC * NWISH, ED), _f32),
        pltpu.VMEM((HC * NTAGS, ED), _f32),
        pltpu.VMEM((HC, ED), _f32),
        pltpu.VMEM((HC, ED), _f32),
        pltpu.VMEM((HC, ED), _f32),
        pltpu.VMEM((HC, ED), _f32),
        pltpu.VMEM((HC, ED), _f32),
        pltpu.SemaphoreType.DMA,
    ],
    compiler_params=pltpu.CompilerParams(use_tc_tiling_on_sc=False),
)(_sc_body)


def _tc_body(u0, ipart, dense, w1, b1, w2, b2, wu1, bu1, wu2, bu2, wu3, bu3,
             out):
    uh = jnp.maximum(
        jnp.dot(u0[...], wu1[...], preferred_element_type=_f32) + bu1[...], 0.0)
    uh = jnp.maximum(
        jnp.dot(uh, wu2[...], preferred_element_type=_f32) + bu2[...], 0.0)
    u_emb = jnp.dot(uh, wu3[...], preferred_element_type=_f32) + bu3[...]
    d = dense[...]
    w1v = w1[...]
    dh = (d[:, 0:1] * w1v[0:1, :] + d[:, 1:2] * w1v[1:2, :]
          + d[:, 2:3] * w1v[2:3, :] + b1[...])
    dh = jnp.maximum(dh, 0.0)
    d_e = jnp.dot(dh, w2[...], preferred_element_type=_f32) + b2[...]
    i_emb = ipart[...] + d_e
    out[...] = jnp.sum(u_emb * i_emb, axis=1, keepdims=True)


def kernel(hist_ids, wish_ids, bid, auth, lang, tags, dense,
           book_emb, auth_emb, lang_emb, tag_emb,
           W1, b1, W2, b2, Wu1, bu1, Wu2, bu2, Wu3, bu3):
    u0, ipart = _sc_gather_pool(
        hist_ids.reshape(-1), wish_ids.reshape(-1), tags.reshape(-1),
        bid, auth, lang, book_emb, auth_emb, lang_emb, tag_emb)
    out = pl.pallas_call(
        _tc_body,
        out_shape=jax.ShapeDtypeStruct((B, 1), _f32),
    )(u0, ipart, dense,
      W1, b1.reshape(1, -1), W2, b2.reshape(1, -1),
      Wu1, bu1.reshape(1, -1), Wu2, bu2.reshape(1, -1),
      Wu3, bu3.reshape(1, -1))
    return out


# speedup vs baseline: 1.2115x; 1.0972x over previous
import functools

import jax
import jax.numpy as jnp
from jax import lax
from jax.experimental import pallas as pl
from jax.experimental.pallas import tpu as pltpu
from jax.experimental.pallas import tpu_sc as plsc

B = 4096
ED = 64
NHIST = 50
NWISH = 20
NTAGS = 10
NLANG = 51
NCORES = 2
NSUB = 16
NW = NCORES * NSUB
ROWS_W = B // NW
HC = 8
NCH = ROWS_W // HC
FIRE = 80

_f32 = jnp.float32


def _sc_body(hist_hbm, wish_hbm, tags_hbm, bid_hbm, auth_hbm,
             book_hbm, aemb_hbm, temb_hbm,
             u0_hbm, ip_hbm,
             vidx_h, vidx_w, vidx_t, vidx_b, vidx_a,
             rows_h, rows_w, rows_t, rows_b, rows_a,
             out_u, out_i, sem):
    wid = lax.axis_index("s") * NCORES + lax.axis_index("c")
    rbase = wid * ROWS_W

    def gather_rows(table, vidx, rows, n):
        for base in range(0, n, FIRE):
            cnt = min(FIRE, n - base)
            for j16 in range(base, base + cnt, 16):
                v = vidx[pl.ds(j16, 16)]
                for l in range(min(16, base + cnt - j16)):
                    pltpu.async_copy(table.at[pl.ds(v[l], 1)],
                                     rows.at[pl.ds(j16 + l, 1)], sem)
            pltpu.make_async_copy(table.at[pl.ds(0, cnt)],
                                  rows.at[pl.ds(base, cnt)], sem).wait()

    def chunk(c, carry):
        pltpu.sync_copy(
            hist_hbm.at[pl.ds((rbase + c * HC) * NHIST, HC * NHIST)], vidx_h)
        pltpu.sync_copy(
            wish_hbm.at[pl.ds((rbase + c * HC) * NWISH, HC * NWISH)], vidx_w)
        pltpu.sync_copy(
            tags_hbm.at[pl.ds((rbase + c * HC) * NTAGS, HC * NTAGS)], vidx_t)
        pltpu.sync_copy(bid_hbm.at[pl.ds(rbase + c * HC, HC)],
                        vidx_b.at[pl.ds(0, HC)])
        pltpu.sync_copy(auth_hbm.at[pl.ds(rbase + c * HC, HC)],
                        vidx_a.at[pl.ds(0, HC)])

        gather_rows(book_hbm, vidx_h, rows_h, HC * NHIST)
        gather_rows(book_hbm, vidx_w, rows_w, HC * NWISH)
        gather_rows(temb_hbm, vidx_t, rows_t, HC * NTAGS)
        gather_rows(book_hbm, vidx_b, rows_b, HC)
        gather_rows(aemb_hbm, vidx_a, rows_a, HC)

        for r in range(HC):
            for g in range(ED // 16):
                s = pl.ds(g * 16, 16)
                acch = rows_h[r * NHIST, s]
                for j in range(1, NHIST):
                    acch = acch + rows_h[r * NHIST + j, s]
                accw = rows_w[r * NWISH, s]
                for j in range(1, NWISH):
                    accw = accw + rows_w[r * NWISH + j, s]
                out_u[r, s] = acch * (1.0 / NHIST) + accw * (1.0 / NWISH)
                acct = rows_t[r * NTAGS, s]
                for j in range(1, NTAGS):
                    acct = acct + rows_t[r * NTAGS + j, s]
                out_i[r, s] = (rows_b[r, s] + rows_a[r, s]
                               + acct * (1.0 / NTAGS))

        pltpu.sync_copy(out_u, u0_hbm.at[pl.ds(rbase + c * HC, HC)])
        pltpu.sync_copy(out_i, ip_hbm.at[pl.ds(rbase + c * HC, HC)])
        return carry

    lax.fori_loop(0, NCH, chunk, 0)


_sc_gather_pool = functools.partial(
    pl.kernel,
    out_type=(jax.ShapeDtypeStruct((B, ED), _f32),
              jax.ShapeDtypeStruct((B, ED), _f32)),
    mesh=plsc.VectorSubcoreMesh(core_axis_name="c", subcore_axis_name="s"),
    scratch_types=[
        pltpu.VMEM((HC * NHIST,), jnp.int32),
        pltpu.VMEM((HC * NWISH,), jnp.int32),
        pltpu.VMEM((HC * NTAGS,), jnp.int32),
        pltpu.VMEM((16,), jnp.int32),
        pltpu.VMEM((16,), jnp.int32),
        pltpu.VMEM((HC * NHIST, ED), _f32),
        pltpu.VMEM((HC * NWISH, ED), _f32),
        pltpu.VMEM((HC * NTAGS, ED), _f32),
        pltpu.VMEM((HC, ED), _f32),
        pltpu.VMEM((HC, ED), _f32),
        pltpu.VMEM((HC, ED), _f32),
        pltpu.VMEM((HC, ED), _f32),
        pltpu.SemaphoreType.DMA,
    ],
    compiler_params=pltpu.CompilerParams(use_tc_tiling_on_sc=True),
)(_sc_body)


def _tc_body(u0, ipart, dense, lang, lemb, w1, b1, w2, b2,
             wu1, bu1, wu2, bu2, wu3, bu3, out):
    uh = jnp.maximum(
        jnp.dot(u0[...], wu1[...], preferred_element_type=_f32) + bu1[...], 0.0)
    uh = jnp.maximum(
        jnp.dot(uh, wu2[...], preferred_element_type=_f32) + bu2[...], 0.0)
    u_emb = jnp.dot(uh, wu3[...], preferred_element_type=_f32) + bu3[...]
    d = dense[...]
    w1v = w1[...]
    dh = (d[:, 0:1] * w1v[0:1, :] + d[:, 1:2] * w1v[1:2, :]
          + d[:, 2:3] * w1v[2:3, :] + b1[...])
    dh = jnp.maximum(dh, 0.0)
    d_e = jnp.dot(dh, w2[...], preferred_element_type=_f32) + b2[...]
    onehot = (lang[...] == lax.broadcasted_iota(jnp.int32, (1, NLANG), 1))
    l_e = jnp.dot(onehot.astype(_f32), lemb[...],
                  preferred_element_type=_f32)
    i_emb = ipart[...] + d_e + l_e
    out[...] = jnp.sum(u_emb * i_emb, axis=1, keepdims=True)


def kernel(hist_ids, wish_ids, bid, auth, lang, tags, dense,
           book_emb, auth_emb, lang_emb, tag_emb,
           W1, b1, W2, b2, Wu1, bu1, Wu2, bu2, Wu3, bu3):
    u0, ipart = _sc_gather_pool(
        hist_ids.reshape(-1), wish_ids.reshape(-1), tags.reshape(-1),
        bid, auth, book_emb, auth_emb, tag_emb)
    out = pl.pallas_call(
        _tc_body,
        out_shape=jax.ShapeDtypeStruct((B, 1), _f32),
    )(u0, ipart, dense, lang.reshape(B, 1), lang_emb,
      W1, b1.reshape(1, -1), W2, b2.reshape(1, -1),
      Wu1, bu1.reshape(1, -1), Wu2, bu2.reshape(1, -1),
      Wu3, bu3.reshape(1, -1))
    return out


# speedup vs baseline: 1.3412x; 1.1071x over previous
import functools

import jax
import jax.numpy as jnp
from jax import lax
from jax.experimental import pallas as pl
from jax.experimental.pallas import tpu as pltpu
from jax.experimental.pallas import tpu_sc as plsc

B = 4096
ED = 64
NHIST = 50
NWISH = 20
NTAGS = 10
NLANG = 51
NCORES = 2
NSUB = 16
NW = NCORES * NSUB
ROWS_W = B // NW
HC = 8
NCH = ROWS_W // HC
FIRE = 80

_f32 = jnp.float32


def _sc_body(hist_hbm, wish_hbm, tags_hbm, bid_hbm, auth_hbm,
             book_hbm, aemb_hbm, temb_hbm,
             u0_hbm, ip_hbm,
             vidx_h, vidx_w, vidx_t, vidx_b, vidx_a,
             rows_h, rows_w, rows_t, rows_b, rows_a,
             out_u, out_i, sem):
    wid = lax.axis_index("s") * NCORES + lax.axis_index("c")
    rbase = wid * ROWS_W

    def gather_rows(table, vidx, rows, n):
        for j16 in range(0, n, 16):
            v = vidx[pl.ds(j16, 16)]
            for l in range(min(16, n - j16)):
                pltpu.async_copy(table.at[pl.ds(v[l], 1)],
                                 rows.at[pl.ds(j16 + l, 1)], sem)

    def drain_rows(table, rows, n):
        pltpu.make_async_copy(table.at[pl.ds(0, n)],
                              rows.at[pl.ds(0, n)], sem).wait()

    def chunk(c, carry):
        pltpu.sync_copy(
            hist_hbm.at[pl.ds((rbase + c * HC) * NHIST, HC * NHIST)], vidx_h)
        pltpu.sync_copy(
            wish_hbm.at[pl.ds((rbase + c * HC) * NWISH, HC * NWISH)], vidx_w)
        pltpu.sync_copy(
            tags_hbm.at[pl.ds((rbase + c * HC) * NTAGS, HC * NTAGS)], vidx_t)
        pltpu.sync_copy(bid_hbm.at[pl.ds(rbase + c * HC, HC)],
                        vidx_b.at[pl.ds(0, HC)])
        pltpu.sync_copy(auth_hbm.at[pl.ds(rbase + c * HC, HC)],
                        vidx_a.at[pl.ds(0, HC)])

        gather_rows(book_hbm, vidx_h, rows_h, HC * NHIST)
        gather_rows(book_hbm, vidx_w, rows_w, HC * NWISH)
        gather_rows(temb_hbm, vidx_t, rows_t, HC * NTAGS)
        gather_rows(book_hbm, vidx_b, rows_b, HC)
        gather_rows(aemb_hbm, vidx_a, rows_a, HC)
        drain_rows(book_hbm, rows_h, HC * NHIST)
        drain_rows(book_hbm, rows_w, HC * NWISH)
        drain_rows(temb_hbm, rows_t, HC * NTAGS)
        drain_rows(book_hbm, rows_b, HC)
        drain_rows(aemb_hbm, rows_a, HC)

        for r in range(HC):
            for g in range(ED // 16):
                s = pl.ds(g * 16, 16)
                acch = rows_h[r * NHIST, s]
                for j in range(1, NHIST):
                    acch = acch + rows_h[r * NHIST + j, s]
                accw = rows_w[r * NWISH, s]
                for j in range(1, NWISH):
                    accw = accw + rows_w[r * NWISH + j, s]
                out_u[r, s] = acch * (1.0 / NHIST) + accw * (1.0 / NWISH)
                acct = rows_t[r * NTAGS, s]
                for j in range(1, NTAGS):
                    acct = acct + rows_t[r * NTAGS + j, s]
                out_i[r, s] = (rows_b[r, s] + rows_a[r, s]
                               + acct * (1.0 / NTAGS))

        pltpu.sync_copy(out_u, u0_hbm.at[pl.ds(rbase + c * HC, HC)])
        pltpu.sync_copy(out_i, ip_hbm.at[pl.ds(rbase + c * HC, HC)])
        return carry

    lax.fori_loop(0, NCH, chunk, 0)


_sc_gather_pool = functools.partial(
    pl.kernel,
    out_type=(jax.ShapeDtypeStruct((B, ED), _f32),
              jax.ShapeDtypeStruct((B, ED), _f32)),
    mesh=plsc.VectorSubcoreMesh(core_axis_name="c", subcore_axis_name="s"),
    scratch_types=[
        pltpu.VMEM((HC * NHIST,), jnp.int32),
        pltpu.VMEM((HC * NWISH,), jnp.int32),
        pltpu.VMEM((HC * NTAGS,), jnp.int32),
        pltpu.VMEM((16,), jnp.int32),
        pltpu.VMEM((16,), jnp.int32),
        pltpu.VMEM((HC * NHIST, ED), _f32),
        pltpu.VMEM((HC * NWISH, ED), _f32),
        pltpu.VMEM((HC * NTAGS, ED), _f32),
        pltpu.VMEM((HC, ED), _f32),
        pltpu.VMEM((HC, ED), _f32),
        pltpu.VMEM((HC, ED), _f32),
        pltpu.VMEM((HC, ED), _f32),
        pltpu.SemaphoreType.DMA,
    ],
    compiler_params=pltpu.CompilerParams(use_tc_tiling_on_sc=True,
                                         needs_layout_passes=True),
)(_sc_body)


def _tc_body(u0, ipart, dense, lang, lemb, w1, b1, w2, b2,
             wu1, bu1, wu2, bu2, wu3, bu3, out):
    uh = jnp.maximum(
        jnp.dot(u0[...], wu1[...], preferred_element_type=_f32) + bu1[...], 0.0)
    uh = jnp.maximum(
        jnp.dot(uh, wu2[...], preferred_element_type=_f32) + bu2[...], 0.0)
    u_emb = jnp.dot(uh, wu3[...], preferred_element_type=_f32) + bu3[...]
    d = dense[...]
    w1v = w1[...]
    dh = (d[:, 0:1] * w1v[0:1, :] + d[:, 1:2] * w1v[1:2, :]
          + d[:, 2:3] * w1v[2:3, :] + b1[...])
    dh = jnp.maximum(dh, 0.0)
    d_e = jnp.dot(dh, w2[...], preferred_element_type=_f32) + b2[...]
    onehot = (lang[...] == lax.broadcasted_iota(jnp.int32, (1, NLANG), 1))
    l_e = jnp.dot(onehot.astype(_f32), lemb[...],
                  preferred_element_type=_f32)
    i_emb = ipart[...] + d_e + l_e
    out[...] = jnp.sum(u_emb * i_emb, axis=1, keepdims=True)


def kernel(hist_ids, wish_ids, bid, auth, lang, tags, dense,
           book_emb, auth_emb, lang_emb, tag_emb,
           W1, b1, W2, b2, Wu1, bu1, Wu2, bu2, Wu3, bu3):
    u0, ipart = _sc_gather_pool(
        hist_ids.reshape(-1), wish_ids.reshape(-1), tags.reshape(-1),
        bid, auth, book_emb, auth_emb, tag_emb)
    out = pl.pallas_call(
        _tc_body,
        out_shape=jax.ShapeDtypeStruct((B, 1), _f32),
    )(u0, ipart, dense, lang.reshape(B, 1), lang_emb,
      W1, b1.reshape(1, -1), W2, b2.reshape(1, -1),
      Wu1, bu1.reshape(1, -1), Wu2, bu2.reshape(1, -1),
      Wu3, bu3.reshape(1, -1))
    return out
